# baseline (speedup 1.0000x reference)
import jax
import jax.numpy as jnp
from jax import lax
from jax.experimental import pallas as pl
from jax.experimental.pallas import tpu as pltpu

IN_DIM = 768
HASHES = 2
BITS = 6
NGROUPS = 2 * HASHES
OUT_LANES = 8
ROWS = 4096


def _router_body(x_ref, wt_ref, aux_ref, out_ref):
    z = jnp.dot(x_ref[...], wt_ref[...], preferred_element_type=jnp.float32)
    bits = (z > 0).astype(jnp.float32)
    j = lax.broadcasted_iota(jnp.int32, (128, OUT_LANES), 0)
    g = lax.broadcasted_iota(jnp.int32, (128, OUT_LANES), 1)
    mask = (j < BITS * NGROUPS) & ((j // BITS) == g)
    s = jnp.where(mask, (1 << (j % BITS)).astype(jnp.float32), 0.0)
    packed = jnp.dot(bits, s, preferred_element_type=jnp.float32)
    idx = packed.astype(jnp.int32) + aux_ref[0, 0]
    out_ref[...] = idx.T


def _router(x, wt, aux):
    n = x.shape[0]
    grid = (n // ROWS,)
    return pl.pallas_call(
        _router_body,
        grid=grid,
        in_specs=[
            pl.BlockSpec((ROWS, IN_DIM), lambda i: (i, 0)),
            pl.BlockSpec((IN_DIM, 128), lambda i: (0, 0)),
            pl.BlockSpec((1, 1), lambda i: (0, 0), memory_space=pltpu.MemorySpace.SMEM),
        ],
        out_specs=pl.BlockSpec((OUT_LANES, ROWS), lambda i: (0, i)),
        out_shape=jax.ShapeDtypeStruct((OUT_LANES, n), jnp.int32),
        compiler_params=pltpu.CompilerParams(
            dimension_semantics=("parallel",),
        ),
    )(x, wt, aux)


def kernel(tag, W_read, W_write, collect_aux=0):
    B, T, D = tag.shape
    x = tag.reshape(B * T, D)
    w = jnp.concatenate([W_read, W_write], axis=0)
    wt = jnp.pad(w.T, ((0, 0), (0, 128 - NGROUPS * BITS)))
    aux = jnp.asarray(collect_aux, dtype=jnp.int32).reshape(1, 1)
    out = _router(x, wt, aux)
    idx_r = out[0:HASHES].T.reshape(B, T, HASHES)
    idx_w = out[HASHES:2 * HASHES].T.reshape(B, T, HASHES)
    return idx_r, idx_w
